# baseline (speedup 1.0000x reference)
import jax
import jax.numpy as jnp
from jax.experimental import pallas as pl
from jax.experimental.pallas import tpu as pltpu

_N_DOMAINS = 8
_EPS = 1e-5
_BT = 512


def _bn_kernel(
    d_ref, xa_ref, xb_ref, gamma_ref, beta_ref, out_ref,
    sums_ref, sumsq_ref, cnt_ref, scale_ref, shift_ref,
):
    p = pl.program_id(0)
    i = pl.program_id(1)
    dvec = d_ref[0, 0, :]
    onehot = (
        dvec[:, None]
        == jax.lax.broadcasted_iota(jnp.int32, (dvec.shape[0], _N_DOMAINS), 1)
    ).astype(jnp.float32)
    dh = xa_ref.shape[1]

    @pl.when(p == 0)
    def _stats():
        xa = xa_ref[...]
        xb = xb_ref[...]
        oT = onehot.T
        sa = jax.lax.dot(oT, xa, preferred_element_type=jnp.float32)
        sb = jax.lax.dot(oT, xb, preferred_element_type=jnp.float32)
        sqa = jax.lax.dot(oT, xa * xa, preferred_element_type=jnp.float32)
        sqb = jax.lax.dot(oT, xb * xb, preferred_element_type=jnp.float32)
        c = jnp.broadcast_to(jnp.sum(onehot, axis=0)[:, None], (_N_DOMAINS, 128))

        @pl.when(i == 0)
        def _():
            sums_ref[:, :dh] = sa
            sums_ref[:, dh:] = sb
            sumsq_ref[:, :dh] = sqa
            sumsq_ref[:, dh:] = sqb
            cnt_ref[...] = c

        @pl.when(i != 0)
        def _():
            sums_ref[:, :dh] += sa
            sums_ref[:, dh:] += sb
            sumsq_ref[:, :dh] += sqa
            sumsq_ref[:, dh:] += sqb
            cnt_ref[...] += c

    @pl.when(p == 1)
    def _apply():
        @pl.when(i == 0)
        def _():
            cnt = jnp.maximum(cnt_ref[:, 0:1], 1.0)
            mean = sums_ref[...] / cnt
            var = jnp.maximum(sumsq_ref[...] / cnt - mean * mean, 0.0)
            scale = gamma_ref[...] * jax.lax.rsqrt(var + _EPS)
            scale_ref[...] = scale
            shift_ref[...] = beta_ref[...] - mean * scale

        sca = jax.lax.dot(onehot, scale_ref[:, :dh], preferred_element_type=jnp.float32)
        scb = jax.lax.dot(onehot, scale_ref[:, dh:], preferred_element_type=jnp.float32)
        sha = jax.lax.dot(onehot, shift_ref[:, :dh], preferred_element_type=jnp.float32)
        shb = jax.lax.dot(onehot, shift_ref[:, dh:], preferred_element_type=jnp.float32)
        out_ref[:, :dh] = xa_ref[...] * sca + sha
        out_ref[:, dh:] = xb_ref[...] * scb + shb


def kernel(X, d, gamma, beta):
    nt, dm = X.shape
    nb = nt // _BT
    d_r = d.reshape(nb, 1, _BT)

    out = pl.pallas_call(
        _bn_kernel,
        grid=(2, nb),
        in_specs=[
            pl.BlockSpec((1, 1, _BT), lambda p, i: (i, 0, 0)),
            pl.BlockSpec((_BT, dm // 2), lambda p, i: (i, 0)),
            pl.BlockSpec((_BT, dm // 2), lambda p, i: (i, 1)),
            pl.BlockSpec((_N_DOMAINS, dm), lambda p, i: (0, 0)),
            pl.BlockSpec((_N_DOMAINS, dm), lambda p, i: (0, 0)),
        ],
        out_specs=pl.BlockSpec((_BT, dm), lambda p, i: (i * p, 0)),
        out_shape=jax.ShapeDtypeStruct((nt, dm), jnp.float32),
        scratch_shapes=[
            pltpu.VMEM((_N_DOMAINS, dm), jnp.float32),
            pltpu.VMEM((_N_DOMAINS, dm), jnp.float32),
            pltpu.VMEM((_N_DOMAINS, 128), jnp.float32),
            pltpu.VMEM((_N_DOMAINS, dm), jnp.float32),
            pltpu.VMEM((_N_DOMAINS, dm), jnp.float32),
        ],
    )(d_r, X, X, gamma, beta)
    return out
